# baseline (speedup 1.0000x reference)
import functools

import jax
import jax.numpy as jnp
from jax import lax
from jax.experimental import pallas as pl
from jax.experimental.pallas import tpu as pltpu
from jax.experimental.pallas import tpu_sc as plsc


def _topk_body(d_ref, idx_ref):
    d = d_ref[0]
    R, P = d.shape
    iota = lax.broadcasted_iota(jnp.int32, (R, P), 1)
    inf = jnp.float32(jnp.inf)
    for r in range(16):
        m = jnp.min(d, axis=1, keepdims=True)
        js = jnp.min(jnp.where(d == m, iota, jnp.int32(P)), axis=1)
        idx_ref[0, r, :] = js
        d = jnp.where(iota == js[:, None], inf, d)


def _topk16(d):
    B, Pp, P = d.shape
    R = min(Pp, 256)
    return pl.pallas_call(
        _topk_body,
        grid=(B, Pp // R),
        in_specs=[pl.BlockSpec((1, R, P), lambda b, p: (b, p, 0))],
        out_specs=pl.BlockSpec((1, 16, R), lambda b, p: (b, 0, p)),
        out_shape=jax.ShapeDtypeStruct((B, 16, Pp), jnp.int32),
    )(d)


def _gather_max_sc(S, idxT, T, n_cs, n_ds):
    B, O, P = S.shape
    Pp = idxT.shape[2]
    CH = O // n_cs
    DP = Pp // n_ds
    pgs = DP // 16
    mesh = plsc.VectorSubcoreMesh(core_axis_name="c", subcore_axis_name="s")

    @functools.partial(
        pl.kernel,
        mesh=mesh,
        compiler_params=pltpu.CompilerParams(
            use_tc_tiling_on_sc=False, needs_layout_passes=False),
        out_type=jax.ShapeDtypeStruct((B, O, Pp), jnp.float32),
        scratch_types=[
            pltpu.VMEM((CH * P,), jnp.float32),
            pltpu.VMEM((16, DP), jnp.int32),
            pltpu.VMEM((CH, DP), jnp.float32),
            pltpu.VMEM((CH, DP), jnp.float32),
        ],
    )
    def k(S_hbm, idx_hbm, T_hbm, out_hbm, S_v, idx_v, T_v, out_v):
        wid = lax.axis_index("s") * 2 + lax.axis_index("c")
        ds = wid // n_cs
        cs = wid % n_cs
        c0 = cs * CH
        p0 = ds * DP
        for b in range(B):
            pltpu.sync_copy(S_hbm.at[b, pl.ds(c0 * P, CH * P)], S_v)
            pltpu.sync_copy(idx_hbm.at[b, :, pl.ds(p0, DP)], idx_v)
            pltpu.sync_copy(T_hbm.at[b, pl.ds(c0, CH), pl.ds(p0, DP)], T_v)

            def pg_body(pg, carry):
                base = pg * 16
                avs = [idx_v[kk, pl.ds(base, 16)] for kk in range(16)]
                for o in range(CH):
                    offs = [a + jnp.int32(o * P) for a in avs]
                    acc = plsc.load_gather(S_v, [offs[0]])
                    for kk in range(1, 16):
                        acc = jnp.maximum(acc, plsc.load_gather(S_v, [offs[kk]]))
                    tv = T_v[o, pl.ds(base, 16)]
                    out_v[o, pl.ds(base, 16)] = jnp.maximum(acc + tv, 0.0)
                return carry

            lax.fori_loop(0, pgs, pg_body, 0)
            pltpu.sync_copy(out_v, out_hbm.at[b, pl.ds(c0, CH), pl.ds(p0, DP)])

    return k(S.reshape(B, O * P), idxT, T)


def _outfc_body(f_ref, w_ref, b_ref, o_ref):
    h = jnp.dot(w_ref[...].T, f_ref[0],
                preferred_element_type=jnp.float32,
                precision=lax.Precision.HIGHEST)
    o_ref[0] = jnp.maximum(h + b_ref[...].T, 0.0)


def _out_fc(f, W, b):
    B, C, Pp = f.shape
    O = W.shape[1]
    return pl.pallas_call(
        _outfc_body,
        grid=(B,),
        in_specs=[
            pl.BlockSpec((1, C, Pp), lambda i: (i, 0, 0)),
            pl.BlockSpec((C, O), lambda i: (0, 0)),
            pl.BlockSpec((1, O), lambda i: (0, 0)),
        ],
        out_specs=pl.BlockSpec((1, O, Pp), lambda i: (i, 0, 0)),
        out_shape=jax.ShapeDtypeStruct((B, O, Pp), jnp.float32),
    )(f, W, b[None, :])


def _layer(xyz, feats, W, b, stride, n_cs, n_ds):
    C = feats.shape[1]
    Wa, Wb, Wc = W[:C], W[C : 2 * C], W[2 * C :]
    q = xyz[:, ::stride]
    d = (jnp.sum(q * q, axis=2)[:, :, None]
         + jnp.sum(xyz * xyz, axis=2)[:, None, :]
         - 2.0 * jnp.matmul(q, jnp.swapaxes(xyz, 1, 2)))
    idxT = _topk16(d)
    hp = lax.Precision.HIGHEST
    S = (jnp.einsum("co,bcp->bop", Wa, feats, precision=hp)
         + jnp.einsum("do,bpd->bop", Wc, xyz, precision=hp))
    T = (jnp.einsum("co,bcp->bop", Wb - Wa, feats[:, :, ::stride], precision=hp)
         - jnp.einsum("do,bpd->bop", Wc, q, precision=hp)
         + b[None, :, None])
    out = _gather_max_sc(S, idxT, T, n_cs, n_ds)
    return q, out


def kernel(xyz, rgb, W_in, b_in, W0, b0, W1, b1, W_out, b_out):
    _, f0 = _layer(xyz, rgb, W_in, b_in, 1, 2, 16)
    xyz1, f1 = _layer(xyz, f0, W0, b0, 4, 4, 8)
    xyz2, f2 = _layer(xyz1, f1, W1, b1, 4, 4, 8)
    f2 = _out_fc(f2, W_out, b_out)
    return (xyz, xyz1, xyz2, f0, f1, f2)


# speedup vs baseline: 1.0533x; 1.0533x over previous
import functools

import jax
import jax.numpy as jnp
from jax import lax
from jax.experimental import pallas as pl
from jax.experimental.pallas import tpu as pltpu
from jax.experimental.pallas import tpu_sc as plsc


def _topk_body(q_ref, sT_ref, q2_ref, s2_ref, idx_ref, d_ref):
    m2 = jnp.dot(q_ref[0], sT_ref[0], preferred_element_type=jnp.float32)
    d_ref[...] = (q2_ref[0, 0][:, None] + s2_ref[0]) - 2.0 * m2
    d = d_ref[...]
    R, P = d.shape
    iota = lax.broadcasted_iota(jnp.int32, (R, P), 1)
    inf = jnp.float32(jnp.inf)
    for r in range(16):
        m = jnp.min(d, axis=1, keepdims=True)
        js = jnp.min(jnp.where(d == m, iota, jnp.int32(P)), axis=1)
        idx_ref[0, r, :] = js
        d = jnp.where(iota == js[:, None], inf, d)


def _topk16(q, s):
    B, Pp, _ = q.shape
    P = s.shape[1]
    R = min(Pp, 256)
    sT = jnp.swapaxes(s, 1, 2)
    q2 = jnp.sum(q * q, axis=2)[:, None, :]
    s2 = jnp.sum(s * s, axis=2)[:, None, :]
    return pl.pallas_call(
        _topk_body,
        grid=(B, Pp // R),
        in_specs=[
            pl.BlockSpec((1, R, 3), lambda b, p: (b, p, 0)),
            pl.BlockSpec((1, 3, P), lambda b, p: (b, 0, 0)),
            pl.BlockSpec((1, 1, R), lambda b, p: (b, 0, p)),
            pl.BlockSpec((1, 1, P), lambda b, p: (b, 0, 0)),
        ],
        out_specs=pl.BlockSpec((1, 16, R), lambda b, p: (b, 0, p)),
        out_shape=jax.ShapeDtypeStruct((B, 16, Pp), jnp.int32),
        scratch_shapes=[pltpu.VMEM((R, P), jnp.float32)],
    )(q, sT, q2, s2)


def _gather_max_sc(S, idxT, T, n_cs, n_ds):
    B, O, P = S.shape
    Pp = idxT.shape[2]
    CH = O // n_cs
    DP = Pp // n_ds
    pgs = DP // 16
    mesh = plsc.VectorSubcoreMesh(core_axis_name="c", subcore_axis_name="s")

    @functools.partial(
        pl.kernel,
        mesh=mesh,
        compiler_params=pltpu.CompilerParams(
            use_tc_tiling_on_sc=False, needs_layout_passes=False),
        out_type=jax.ShapeDtypeStruct((B, O, Pp), jnp.float32),
        scratch_types=[
            pltpu.VMEM((CH * P,), jnp.float32),
            pltpu.VMEM((16, DP), jnp.int32),
            pltpu.VMEM((CH, DP), jnp.float32),
            pltpu.VMEM((CH, DP), jnp.float32),
        ],
    )
    def k(S_hbm, idx_hbm, T_hbm, out_hbm, S_v, idx_v, T_v, out_v):
        wid = lax.axis_index("s") * 2 + lax.axis_index("c")
        ds = wid // n_cs
        cs = wid % n_cs
        c0 = cs * CH
        p0 = ds * DP
        for b in range(B):
            pltpu.sync_copy(S_hbm.at[b, pl.ds(c0 * P, CH * P)], S_v)
            pltpu.sync_copy(idx_hbm.at[b, :, pl.ds(p0, DP)], idx_v)
            pltpu.sync_copy(T_hbm.at[b, pl.ds(c0, CH), pl.ds(p0, DP)], T_v)

            def pg_body(pg, carry):
                base = pg * 16
                avs = [idx_v[kk, pl.ds(base, 16)] for kk in range(16)]
                for o in range(CH):
                    offs = [a + jnp.int32(o * P) for a in avs]
                    acc = plsc.load_gather(S_v, [offs[0]])
                    for kk in range(1, 16):
                        acc = jnp.maximum(acc, plsc.load_gather(S_v, [offs[kk]]))
                    tv = T_v[o, pl.ds(base, 16)]
                    out_v[o, pl.ds(base, 16)] = jnp.maximum(acc + tv, 0.0)
                return carry

            lax.fori_loop(0, pgs, pg_body, 0)
            pltpu.sync_copy(out_v, out_hbm.at[b, pl.ds(c0, CH), pl.ds(p0, DP)])

    return k(S.reshape(B, O * P), idxT, T)


def _outfc_body(f_ref, w_ref, b_ref, o_ref):
    h = jnp.dot(w_ref[...].T, f_ref[0],
                preferred_element_type=jnp.float32,
                precision=lax.Precision.HIGHEST)
    o_ref[0] = jnp.maximum(h + b_ref[...].T, 0.0)


def _out_fc(f, W, b):
    B, C, Pp = f.shape
    O = W.shape[1]
    return pl.pallas_call(
        _outfc_body,
        grid=(B,),
        in_specs=[
            pl.BlockSpec((1, C, Pp), lambda i: (i, 0, 0)),
            pl.BlockSpec((C, O), lambda i: (0, 0)),
            pl.BlockSpec((1, O), lambda i: (0, 0)),
        ],
        out_specs=pl.BlockSpec((1, O, Pp), lambda i: (i, 0, 0)),
        out_shape=jax.ShapeDtypeStruct((B, O, Pp), jnp.float32),
    )(f, W, b[None, :])


def _layer(xyz, feats, W, b, stride, n_cs, n_ds):
    C = feats.shape[1]
    Wa, Wb, Wc = W[:C], W[C : 2 * C], W[2 * C :]
    q = xyz[:, ::stride]
    idxT = _topk16(q, xyz)
    hp = lax.Precision.HIGHEST
    S = (jnp.einsum("co,bcp->bop", Wa, feats, precision=hp)
         + jnp.einsum("do,bpd->bop", Wc, xyz, precision=hp))
    T = (jnp.einsum("co,bcp->bop", Wb - Wa, feats[:, :, ::stride], precision=hp)
         - jnp.einsum("do,bpd->bop", Wc, q, precision=hp)
         + b[None, :, None])
    out = _gather_max_sc(S, idxT, T, n_cs, n_ds)
    return q, out


def kernel(xyz, rgb, W_in, b_in, W0, b0, W1, b1, W_out, b_out):
    _, f0 = _layer(xyz, rgb, W_in, b_in, 1, 2, 16)
    xyz1, f1 = _layer(xyz, f0, W0, b0, 4, 4, 8)
    xyz2, f2 = _layer(xyz1, f1, W1, b1, 4, 4, 8)
    f2 = _out_fc(f2, W_out, b_out)
    return (xyz, xyz1, xyz2, f0, f1, f2)


# speedup vs baseline: 1.1600x; 1.1013x over previous
import functools

import jax
import jax.numpy as jnp
from jax import lax
from jax.experimental import pallas as pl
from jax.experimental.pallas import tpu as pltpu
from jax.experimental.pallas import tpu_sc as plsc


def _topk_body(q_ref, sT_ref, q2_ref, s2_ref, idx_ref, d_ref):
    m2 = jnp.dot(q_ref[0], sT_ref[0], preferred_element_type=jnp.float32)
    d_ref[...] = (q2_ref[0, 0][:, None] + s2_ref[0]) - 2.0 * m2
    d = d_ref[...]
    R, P = d.shape
    iota = lax.broadcasted_iota(jnp.int32, (R, P), 1)
    inf = jnp.float32(jnp.inf)
    for r in range(16):
        js = jnp.argmin(d, axis=1).astype(jnp.int32)
        idx_ref[0, r, :] = js
        d = jnp.where(iota == js[:, None], inf, d)


def _topk16(q, s):
    B, Pp, _ = q.shape
    P = s.shape[1]
    R = min(Pp, 256)
    sT = jnp.swapaxes(s, 1, 2)
    q2 = jnp.sum(q * q, axis=2)[:, None, :]
    s2 = jnp.sum(s * s, axis=2)[:, None, :]
    return pl.pallas_call(
        _topk_body,
        grid=(B, Pp // R),
        in_specs=[
            pl.BlockSpec((1, R, 3), lambda b, p: (b, p, 0)),
            pl.BlockSpec((1, 3, P), lambda b, p: (b, 0, 0)),
            pl.BlockSpec((1, 1, R), lambda b, p: (b, 0, p)),
            pl.BlockSpec((1, 1, P), lambda b, p: (b, 0, 0)),
        ],
        out_specs=pl.BlockSpec((1, 16, R), lambda b, p: (b, 0, p)),
        out_shape=jax.ShapeDtypeStruct((B, 16, Pp), jnp.int32),
        scratch_shapes=[pltpu.VMEM((R, P), jnp.float32)],
    )(q, sT, q2, s2)


def _gather_max_sc(S, idxT, T, n_cs, n_ds):
    B, O, P = S.shape
    Pp = idxT.shape[2]
    CH = O // n_cs
    DP = Pp // n_ds
    pgs = DP // 16
    mesh = plsc.VectorSubcoreMesh(core_axis_name="c", subcore_axis_name="s")

    @functools.partial(
        pl.kernel,
        mesh=mesh,
        compiler_params=pltpu.CompilerParams(
            use_tc_tiling_on_sc=False, needs_layout_passes=False),
        out_type=jax.ShapeDtypeStruct((B, O, Pp), jnp.float32),
        scratch_types=[
            pltpu.VMEM((CH * P,), jnp.float32),
            pltpu.VMEM((16, DP), jnp.int32),
            pltpu.VMEM((CH, DP), jnp.float32),
            pltpu.VMEM((CH, DP), jnp.float32),
        ],
    )
    def k(S_hbm, idx_hbm, T_hbm, out_hbm, S_v, idx_v, T_v, out_v):
        wid = lax.axis_index("s") * 2 + lax.axis_index("c")
        ds = wid // n_cs
        cs = wid % n_cs
        c0 = cs * CH
        p0 = ds * DP
        for b in range(B):
            pltpu.sync_copy(S_hbm.at[b, pl.ds(c0 * P, CH * P)], S_v)
            pltpu.sync_copy(idx_hbm.at[b, :, pl.ds(p0, DP)], idx_v)
            pltpu.sync_copy(T_hbm.at[b, pl.ds(c0, CH), pl.ds(p0, DP)], T_v)

            def pg_body(pg, carry):
                base = pg * 16
                avs = [idx_v[kk, pl.ds(base, 16)] for kk in range(16)]
                for o in range(CH):
                    offs = [a + jnp.int32(o * P) for a in avs]
                    acc = plsc.load_gather(S_v, [offs[0]])
                    for kk in range(1, 16):
                        acc = jnp.maximum(acc, plsc.load_gather(S_v, [offs[kk]]))
                    tv = T_v[o, pl.ds(base, 16)]
                    out_v[o, pl.ds(base, 16)] = jnp.maximum(acc + tv, 0.0)
                return carry

            lax.fori_loop(0, pgs, pg_body, 0)
            pltpu.sync_copy(out_v, out_hbm.at[b, pl.ds(c0, CH), pl.ds(p0, DP)])

    return k(S.reshape(B, O * P), idxT, T)


def _outfc_body(f_ref, w_ref, b_ref, o_ref):
    h = jnp.dot(w_ref[...].T, f_ref[0],
                preferred_element_type=jnp.float32,
                precision=lax.Precision.HIGHEST)
    o_ref[0] = jnp.maximum(h + b_ref[...].T, 0.0)


def _out_fc(f, W, b):
    B, C, Pp = f.shape
    O = W.shape[1]
    return pl.pallas_call(
        _outfc_body,
        grid=(B,),
        in_specs=[
            pl.BlockSpec((1, C, Pp), lambda i: (i, 0, 0)),
            pl.BlockSpec((C, O), lambda i: (0, 0)),
            pl.BlockSpec((1, O), lambda i: (0, 0)),
        ],
        out_specs=pl.BlockSpec((1, O, Pp), lambda i: (i, 0, 0)),
        out_shape=jax.ShapeDtypeStruct((B, O, Pp), jnp.float32),
    )(f, W, b[None, :])


def _layer(xyz, feats, W, b, stride, n_cs, n_ds):
    C = feats.shape[1]
    Wa, Wb, Wc = W[:C], W[C : 2 * C], W[2 * C :]
    q = xyz[:, ::stride]
    idxT = _topk16(q, xyz)
    hp = lax.Precision.HIGHEST
    S = (jnp.einsum("co,bcp->bop", Wa, feats, precision=hp)
         + jnp.einsum("do,bpd->bop", Wc, xyz, precision=hp))
    T = (jnp.einsum("co,bcp->bop", Wb - Wa, feats[:, :, ::stride], precision=hp)
         - jnp.einsum("do,bpd->bop", Wc, q, precision=hp)
         + b[None, :, None])
    out = _gather_max_sc(S, idxT, T, n_cs, n_ds)
    return q, out


def kernel(xyz, rgb, W_in, b_in, W0, b0, W1, b1, W_out, b_out):
    _, f0 = _layer(xyz, rgb, W_in, b_in, 1, 2, 16)
    xyz1, f1 = _layer(xyz, f0, W0, b0, 4, 4, 8)
    xyz2, f2 = _layer(xyz1, f1, W1, b1, 4, 4, 8)
    f2 = _out_fc(f2, W_out, b_out)
    return (xyz, xyz1, xyz2, f0, f1, f2)
